# baseline (speedup 1.0000x reference)
import functools

import jax
import jax.numpy as jnp
from jax import lax
from jax.experimental import pallas as pl
from jax.experimental.pallas import tpu as pltpu, tpu_sc as plsc

N = 10000
D = 128
E = 320000
EPS = 1e-5

NC = 2
NS = 16
NW = NC * NS
CHUNK = 96
ST0 = 66
ST1 = 143
TOTSTEPS = NS * (ST0 + ST1)
EPAD = TOTSTEPS * CHUNK
RPT = -(-(N + 1) // (NS * 8)) * 8
NPAD = RPT * NS


@functools.cache
def _sc_mesh():
    return plsc.VectorSubcoreMesh(
        core_axis_name="c", subcore_axis_name="s",
        num_cores=NC, num_subcores=NS)


@functools.cache
def _sc_aggregate_kernel():
    return pl.kernel(
        _sc_aggregate_body,
        out_type=jax.ShapeDtypeStruct((NC, NPAD, D), jnp.float32),
        mesh=_sc_mesh(),
        scratch_types=[
            pltpu.VMEM((8, CHUNK), jnp.int32),
            pltpu.VMEM((8, CHUNK), jnp.int32),
            pltpu.VMEM((3, CHUNK, D), jnp.float32),
            pltpu.VMEM_SHARED((NPAD, D), jnp.float32),
            pltpu.SemaphoreType.DMA,
            pltpu.SemaphoreType.DMA,
            pltpu.SemaphoreType.DMA((3,)),
        ],
    )


def _sc_aggregate_body(y_hbm, src_hbm, dst_hbm, zeros_hbm, out_hbm,
                       src_v, dst_v, rows_v, acc_sh, isem, gsem, ssem):
    c = lax.axis_index("c")
    s = lax.axis_index("s")
    base = jnp.where(c == 0, s * ST0, NS * ST0 + s * ST1)
    nst = jnp.where(c == 0, ST0, ST1)

    pltpu.sync_copy(zeros_hbm.at[pl.ds(s * RPT, RPT)],
                    acc_sh.at[pl.ds(s * RPT, RPT)])
    pltpu.sync_copy(src_hbm.at[base], src_v.at[0])
    pltpu.sync_copy(dst_hbm.at[base], dst_v.at[0])
    pltpu.sync_copy(src_hbm.at[base + 1], src_v.at[1])
    pltpu.sync_copy(dst_hbm.at[base + 1], dst_v.at[1])
    plsc.subcore_barrier()
    pltpu.async_copy(y_hbm.at[src_v.at[0]], rows_v.at[0], gsem)

    def body(j, carry):
        buf = lax.rem(j, 3)

        @pl.when(j + 2 < nst)
        def _():
            slot = lax.rem(j + 2, 8)
            pltpu.async_copy(src_hbm.at[base + j + 2], src_v.at[slot], isem)
            pltpu.async_copy(dst_hbm.at[base + j + 2], dst_v.at[slot], isem)

        @pl.when(j >= 2)
        def _():
            jo = j - 2
            slot3 = lax.rem(jo, 3)
            pltpu.make_async_copy(
                rows_v.at[slot3],
                acc_sh.at[dst_v.at[lax.rem(jo, 8)]], ssem.at[slot3]).wait()

        pltpu.make_async_copy(y_hbm.at[src_v.at[lax.rem(j, 8)]],
                              rows_v.at[buf], gsem).wait()

        @pl.when(j + 1 < nst)
        def _():
            pltpu.async_copy(y_hbm.at[src_v.at[lax.rem(j + 1, 8)]],
                             rows_v.at[lax.rem(j + 1, 3)], gsem)

        pltpu.async_copy(rows_v.at[buf], acc_sh.at[dst_v.at[lax.rem(j, 8)]],
                         ssem.at[buf], add=True)

        @pl.when(j + 2 < nst)
        def _():
            slot = lax.rem(j + 2, 8)
            pltpu.make_async_copy(src_hbm.at[base + j + 2], src_v.at[slot],
                                  isem).wait()
            pltpu.make_async_copy(dst_hbm.at[base + j + 2], dst_v.at[slot],
                                  isem).wait()

        return carry

    lax.fori_loop(0, nst, body, 0)
    for dd in (2, 1):
        jo = nst - dd
        pltpu.make_async_copy(
            rows_v.at[lax.rem(jo, 3)],
            acc_sh.at[dst_v.at[lax.rem(jo, 8)]], ssem.at[lax.rem(jo, 3)]).wait()
    plsc.subcore_barrier()
    pltpu.sync_copy(acc_sh.at[pl.ds(s * RPT, RPT)],
                    out_hbm.at[c, pl.ds(s * RPT, RPT)])


def _tc_first_body(x_ref, degs_ref, w_ref, y_ref, dinv_ref):
    deg = degs_ref[0, :N, 0] + degs_ref[1, :N, 0] + 1.0
    dinv = lax.rsqrt(deg)
    dinv_ref[...] = dinv[:, None]
    y_ref[...] = dinv[:, None] * jnp.dot(x_ref[...], w_ref[...],
                                         preferred_element_type=jnp.float32)


def _tc_mid_body(s_ref, y_ref, dinv_ref, b_ref, w_ref, o_ref):
    dinv = dinv_ref[...]
    agg = s_ref[0, :N, :] + s_ref[1, :N, :] + y_ref[...]
    z = agg * dinv + b_ref[...]
    mean = jnp.mean(z, axis=0, keepdims=True)
    var = jnp.mean((z - mean) ** 2, axis=0, keepdims=True)
    t = jnp.maximum((z - mean) / jnp.sqrt(var + EPS), 0.0)
    o_ref[...] = dinv * jnp.dot(t, w_ref[...],
                                preferred_element_type=jnp.float32)


def _tc_head_body(s_ref, y_ref, dinv_ref, b_ref, wl1_ref, bl1_ref, wl2_ref,
                  bl2_ref, o_ref):
    dinv = dinv_ref[...]
    agg = s_ref[0, :N, :] + s_ref[1, :N, :] + y_ref[...]
    z = agg * dinv + b_ref[...]
    mean = jnp.mean(z, axis=0, keepdims=True)
    var = jnp.mean((z - mean) ** 2, axis=0, keepdims=True)
    t = jnp.maximum((z - mean) / jnp.sqrt(var + EPS), 0.0)
    h = jnp.dot(t, wl1_ref[...],
                preferred_element_type=jnp.float32) + bl1_ref[...]
    mean = jnp.mean(h, axis=0, keepdims=True)
    var = jnp.mean((h - mean) ** 2, axis=0, keepdims=True)
    t = jnp.maximum((h - mean) / jnp.sqrt(var + EPS), 0.0)
    o_ref[...] = jnp.dot(t, wl2_ref[...],
                         preferred_element_type=jnp.float32) + bl2_ref[...]


_tc_first = pl.pallas_call(
    _tc_first_body, out_shape=(jax.ShapeDtypeStruct((N, D), jnp.float32),
                               jax.ShapeDtypeStruct((N, 1), jnp.float32)))
_tc_mid = pl.pallas_call(
    _tc_mid_body, out_shape=jax.ShapeDtypeStruct((N, D), jnp.float32))
_tc_head = pl.pallas_call(
    _tc_head_body, out_shape=jax.ShapeDtypeStruct((N, 2), jnp.float32))


def kernel(x, edge_index, W0, b0, W1, b1, W2, b2, Wl1, bl1, Wl2, bl2):
    src = edge_index[0]
    dst = edge_index[1]
    pad = EPAD - E
    src_p = jnp.concatenate([src, jnp.zeros((pad,), jnp.int32)])
    dst_p = jnp.concatenate([dst, jnp.full((pad,), N, jnp.int32)])
    src_r = src_p.reshape(TOTSTEPS, CHUNK)
    dst_r = dst_p.reshape(TOTSTEPS, CHUNK)

    ones_nd = jnp.ones((N, D), jnp.float32)
    zeros = jnp.zeros((NPAD, D), jnp.float32)
    b0r, b1r, b2r = b0[None, :], b1[None, :], b2[None, :]
    bl1r, bl2r = bl1[None, :], bl2[None, :]

    sc_aggregate = _sc_aggregate_kernel()

    degS = sc_aggregate(ones_nd, src_r, dst_r, zeros)
    y, dinv = _tc_first(x, degS, W0)
    S = sc_aggregate(y, src_r, dst_r, zeros)
    y = _tc_mid(S, y, dinv, b0r, W1)
    S = sc_aggregate(y, src_r, dst_r, zeros)
    y = _tc_mid(S, y, dinv, b1r, W2)
    S = sc_aggregate(y, src_r, dst_r, zeros)
    out = _tc_head(S, y, dinv, b2r, Wl1, bl1r, Wl2, bl2r)
    return out


# speedup vs baseline: 1.1306x; 1.1306x over previous
import functools

import jax
import jax.numpy as jnp
from jax import lax
from jax.experimental import pallas as pl
from jax.experimental.pallas import tpu as pltpu, tpu_sc as plsc

N = 10000
D = 128
E = 320000
EPS = 1e-5

NC = 2
NS = 16
NW = NC * NS
CHUNK = 96
ST0 = 92
ST1 = 117
TOTSTEPS = NS * (ST0 + ST1)
EPAD = TOTSTEPS * CHUNK
RPT = -(-(N + 1) // (NS * 8)) * 8
NPAD = RPT * NS


@functools.cache
def _sc_mesh():
    return plsc.VectorSubcoreMesh(
        core_axis_name="c", subcore_axis_name="s",
        num_cores=NC, num_subcores=NS)


@functools.cache
def _sc_aggregate_kernel():
    return pl.kernel(
        _sc_aggregate_body,
        out_type=jax.ShapeDtypeStruct((NC, NPAD, D), jnp.float32),
        mesh=_sc_mesh(),
        scratch_types=[
            pltpu.VMEM((8, CHUNK), jnp.int32),
            pltpu.VMEM((8, CHUNK), jnp.int32),
            pltpu.VMEM((3, CHUNK, D), jnp.float32),
            pltpu.VMEM_SHARED((NPAD, D), jnp.float32),
            pltpu.SemaphoreType.DMA,
            pltpu.SemaphoreType.DMA,
            pltpu.SemaphoreType.DMA((3,)),
        ],
    )


def _sc_aggregate_body(y_hbm, src_hbm, dst_hbm, zeros_hbm, out_hbm,
                       src_v, dst_v, rows_v, acc_sh, isem, gsem, ssem):
    c = lax.axis_index("c")
    s = lax.axis_index("s")
    base = jnp.where(c == 0, s * ST0, NS * ST0 + s * ST1)
    nst = jnp.where(c == 0, ST0, ST1)

    pltpu.sync_copy(zeros_hbm.at[pl.ds(s * RPT, RPT)],
                    acc_sh.at[pl.ds(s * RPT, RPT)])
    pltpu.sync_copy(src_hbm.at[base], src_v.at[0])
    pltpu.sync_copy(dst_hbm.at[base], dst_v.at[0])
    pltpu.sync_copy(src_hbm.at[base + 1], src_v.at[1])
    pltpu.sync_copy(dst_hbm.at[base + 1], dst_v.at[1])
    plsc.subcore_barrier()
    pltpu.async_copy(y_hbm.at[src_v.at[0]], rows_v.at[0], gsem)

    def body(j, carry):
        buf = lax.rem(j, 3)

        @pl.when(j + 2 < nst)
        def _():
            slot = lax.rem(j + 2, 8)
            pltpu.async_copy(src_hbm.at[base + j + 2], src_v.at[slot], isem)
            pltpu.async_copy(dst_hbm.at[base + j + 2], dst_v.at[slot], isem)

        @pl.when(j >= 2)
        def _():
            jo = j - 2
            slot3 = lax.rem(jo, 3)
            pltpu.make_async_copy(
                rows_v.at[slot3],
                acc_sh.at[dst_v.at[lax.rem(jo, 8)]], ssem.at[slot3]).wait()

        pltpu.make_async_copy(y_hbm.at[src_v.at[lax.rem(j, 8)]],
                              rows_v.at[buf], gsem).wait()

        @pl.when(j + 1 < nst)
        def _():
            pltpu.async_copy(y_hbm.at[src_v.at[lax.rem(j + 1, 8)]],
                             rows_v.at[lax.rem(j + 1, 3)], gsem)

        pltpu.async_copy(rows_v.at[buf], acc_sh.at[dst_v.at[lax.rem(j, 8)]],
                         ssem.at[buf], add=True)

        @pl.when(j + 2 < nst)
        def _():
            slot = lax.rem(j + 2, 8)
            pltpu.make_async_copy(src_hbm.at[base + j + 2], src_v.at[slot],
                                  isem).wait()
            pltpu.make_async_copy(dst_hbm.at[base + j + 2], dst_v.at[slot],
                                  isem).wait()

        return carry

    lax.fori_loop(0, nst, body, 0)
    for dd in (2, 1):
        jo = nst - dd
        pltpu.make_async_copy(
            rows_v.at[lax.rem(jo, 3)],
            acc_sh.at[dst_v.at[lax.rem(jo, 8)]], ssem.at[lax.rem(jo, 3)]).wait()
    plsc.subcore_barrier()
    pltpu.sync_copy(acc_sh.at[pl.ds(s * RPT, RPT)],
                    out_hbm.at[c, pl.ds(s * RPT, RPT)])


def _tc_first_body(x_ref, degs_ref, w_ref, y_ref, dinv_ref):
    deg = degs_ref[0, :N, 0] + degs_ref[1, :N, 0] + 1.0
    dinv = lax.rsqrt(deg)
    dinv_ref[...] = dinv[:, None]
    y_ref[...] = dinv[:, None] * jnp.dot(x_ref[...], w_ref[...],
                                         preferred_element_type=jnp.float32)


def _tc_mid_body(s_ref, y_ref, dinv_ref, b_ref, w_ref, o_ref):
    dinv = dinv_ref[...]
    agg = s_ref[0, :N, :] + s_ref[1, :N, :] + y_ref[...]
    z = agg * dinv + b_ref[...]
    mean = jnp.mean(z, axis=0, keepdims=True)
    var = jnp.mean((z - mean) ** 2, axis=0, keepdims=True)
    t = jnp.maximum((z - mean) / jnp.sqrt(var + EPS), 0.0)
    o_ref[...] = dinv * jnp.dot(t, w_ref[...],
                                preferred_element_type=jnp.float32)


def _tc_head_body(s_ref, y_ref, dinv_ref, b_ref, wl1_ref, bl1_ref, wl2_ref,
                  bl2_ref, o_ref):
    dinv = dinv_ref[...]
    agg = s_ref[0, :N, :] + s_ref[1, :N, :] + y_ref[...]
    z = agg * dinv + b_ref[...]
    mean = jnp.mean(z, axis=0, keepdims=True)
    var = jnp.mean((z - mean) ** 2, axis=0, keepdims=True)
    t = jnp.maximum((z - mean) / jnp.sqrt(var + EPS), 0.0)
    h = jnp.dot(t, wl1_ref[...],
                preferred_element_type=jnp.float32) + bl1_ref[...]
    mean = jnp.mean(h, axis=0, keepdims=True)
    var = jnp.mean((h - mean) ** 2, axis=0, keepdims=True)
    t = jnp.maximum((h - mean) / jnp.sqrt(var + EPS), 0.0)
    o_ref[...] = jnp.dot(t, wl2_ref[...],
                         preferred_element_type=jnp.float32) + bl2_ref[...]


_tc_first = pl.pallas_call(
    _tc_first_body, out_shape=(jax.ShapeDtypeStruct((N, D), jnp.float32),
                               jax.ShapeDtypeStruct((N, 1), jnp.float32)))
_tc_mid = pl.pallas_call(
    _tc_mid_body, out_shape=jax.ShapeDtypeStruct((N, D), jnp.float32))
_tc_head = pl.pallas_call(
    _tc_head_body, out_shape=jax.ShapeDtypeStruct((N, 2), jnp.float32))


def kernel(x, edge_index, W0, b0, W1, b1, W2, b2, Wl1, bl1, Wl2, bl2):
    src = edge_index[0]
    dst = edge_index[1]
    pad = EPAD - E
    src_p = jnp.concatenate([src, jnp.zeros((pad,), jnp.int32)])
    dst_p = jnp.concatenate([dst, jnp.full((pad,), N, jnp.int32)])
    src_r = src_p.reshape(TOTSTEPS, CHUNK)
    dst_r = dst_p.reshape(TOTSTEPS, CHUNK)

    ones_nd = jnp.ones((N, D), jnp.float32)
    zeros = jnp.zeros((NPAD, D), jnp.float32)
    b0r, b1r, b2r = b0[None, :], b1[None, :], b2[None, :]
    bl1r, bl2r = bl1[None, :], bl2[None, :]

    sc_aggregate = _sc_aggregate_kernel()

    degS = sc_aggregate(ones_nd, src_r, dst_r, zeros)
    y, dinv = _tc_first(x, degS, W0)
    S = sc_aggregate(y, src_r, dst_r, zeros)
    y = _tc_mid(S, y, dinv, b0r, W1)
    S = sc_aggregate(y, src_r, dst_r, zeros)
    y = _tc_mid(S, y, dinv, b1r, W2)
    S = sc_aggregate(y, src_r, dst_r, zeros)
    out = _tc_head(S, y, dinv, b2r, Wl1, bl1r, Wl2, bl2r)
    return out


# speedup vs baseline: 1.1649x; 1.0303x over previous
import functools

import jax
import jax.numpy as jnp
from jax import lax
from jax.experimental import pallas as pl
from jax.experimental.pallas import tpu as pltpu, tpu_sc as plsc

N = 10000
D = 128
E = 320000
EPS = 1e-5

NC = 2
NS = 16
NW = NC * NS
CHUNK = 96
ST0 = 100
ST1 = 109
TOTSTEPS = NS * (ST0 + ST1)
EPAD = TOTSTEPS * CHUNK
RPT = -(-(N + 1) // (NS * 8)) * 8
NPAD = RPT * NS


@functools.cache
def _sc_mesh():
    return plsc.VectorSubcoreMesh(
        core_axis_name="c", subcore_axis_name="s",
        num_cores=NC, num_subcores=NS)


@functools.cache
def _sc_aggregate_kernel():
    return pl.kernel(
        _sc_aggregate_body,
        out_type=jax.ShapeDtypeStruct((NC, NPAD, D), jnp.float32),
        mesh=_sc_mesh(),
        scratch_types=[
            pltpu.VMEM((8, CHUNK), jnp.int32),
            pltpu.VMEM((8, CHUNK), jnp.int32),
            pltpu.VMEM((3, CHUNK, D), jnp.float32),
            pltpu.VMEM_SHARED((NPAD, D), jnp.float32),
            pltpu.SemaphoreType.DMA,
            pltpu.SemaphoreType.DMA,
            pltpu.SemaphoreType.DMA((3,)),
        ],
    )


def _sc_aggregate_body(y_hbm, src_hbm, dst_hbm, zeros_hbm, out_hbm,
                       src_v, dst_v, rows_v, acc_sh, isem, gsem, ssem):
    c = lax.axis_index("c")
    s = lax.axis_index("s")
    base = jnp.where(c == 0, s * ST0, NS * ST0 + s * ST1)
    nst = jnp.where(c == 0, ST0, ST1)

    pltpu.sync_copy(zeros_hbm.at[pl.ds(s * RPT, RPT)],
                    acc_sh.at[pl.ds(s * RPT, RPT)])
    pltpu.sync_copy(src_hbm.at[base], src_v.at[0])
    pltpu.sync_copy(dst_hbm.at[base], dst_v.at[0])
    pltpu.sync_copy(src_hbm.at[base + 1], src_v.at[1])
    pltpu.sync_copy(dst_hbm.at[base + 1], dst_v.at[1])
    plsc.subcore_barrier()
    pltpu.async_copy(y_hbm.at[src_v.at[0]], rows_v.at[0], gsem)

    def body(j, carry):
        buf = lax.rem(j, 3)

        @pl.when(j + 2 < nst)
        def _():
            slot = lax.rem(j + 2, 8)
            pltpu.async_copy(src_hbm.at[base + j + 2], src_v.at[slot], isem)
            pltpu.async_copy(dst_hbm.at[base + j + 2], dst_v.at[slot], isem)

        @pl.when(j >= 2)
        def _():
            jo = j - 2
            slot3 = lax.rem(jo, 3)
            pltpu.make_async_copy(
                rows_v.at[slot3],
                acc_sh.at[dst_v.at[lax.rem(jo, 8)]], ssem.at[slot3]).wait()

        pltpu.make_async_copy(y_hbm.at[src_v.at[lax.rem(j, 8)]],
                              rows_v.at[buf], gsem).wait()

        @pl.when(j + 1 < nst)
        def _():
            pltpu.async_copy(y_hbm.at[src_v.at[lax.rem(j + 1, 8)]],
                             rows_v.at[lax.rem(j + 1, 3)], gsem)

        pltpu.async_copy(rows_v.at[buf], acc_sh.at[dst_v.at[lax.rem(j, 8)]],
                         ssem.at[buf], add=True)

        @pl.when(j + 2 < nst)
        def _():
            slot = lax.rem(j + 2, 8)
            pltpu.make_async_copy(src_hbm.at[base + j + 2], src_v.at[slot],
                                  isem).wait()
            pltpu.make_async_copy(dst_hbm.at[base + j + 2], dst_v.at[slot],
                                  isem).wait()

        return carry

    lax.fori_loop(0, nst, body, 0)
    for dd in (2, 1):
        jo = nst - dd
        pltpu.make_async_copy(
            rows_v.at[lax.rem(jo, 3)],
            acc_sh.at[dst_v.at[lax.rem(jo, 8)]], ssem.at[lax.rem(jo, 3)]).wait()
    plsc.subcore_barrier()
    pltpu.sync_copy(acc_sh.at[pl.ds(s * RPT, RPT)],
                    out_hbm.at[c, pl.ds(s * RPT, RPT)])


def _tc_first_body(x_ref, degs_ref, w_ref, y_ref, dinv_ref):
    deg = degs_ref[0, :N, 0] + degs_ref[1, :N, 0] + 1.0
    dinv = lax.rsqrt(deg)
    dinv_ref[...] = dinv[:, None]
    y_ref[...] = dinv[:, None] * jnp.dot(x_ref[...], w_ref[...],
                                         preferred_element_type=jnp.float32)


def _tc_mid_body(s_ref, y_ref, dinv_ref, b_ref, w_ref, o_ref):
    dinv = dinv_ref[...]
    agg = s_ref[0, :N, :] + s_ref[1, :N, :] + y_ref[...]
    z = agg * dinv + b_ref[...]
    mean = jnp.mean(z, axis=0, keepdims=True)
    var = jnp.mean((z - mean) ** 2, axis=0, keepdims=True)
    t = jnp.maximum((z - mean) / jnp.sqrt(var + EPS), 0.0)
    o_ref[...] = dinv * jnp.dot(t, w_ref[...],
                                preferred_element_type=jnp.float32)


def _tc_head_body(s_ref, y_ref, dinv_ref, b_ref, wl1_ref, bl1_ref, wl2_ref,
                  bl2_ref, o_ref):
    dinv = dinv_ref[...]
    agg = s_ref[0, :N, :] + s_ref[1, :N, :] + y_ref[...]
    z = agg * dinv + b_ref[...]
    mean = jnp.mean(z, axis=0, keepdims=True)
    var = jnp.mean((z - mean) ** 2, axis=0, keepdims=True)
    t = jnp.maximum((z - mean) / jnp.sqrt(var + EPS), 0.0)
    h = jnp.dot(t, wl1_ref[...],
                preferred_element_type=jnp.float32) + bl1_ref[...]
    mean = jnp.mean(h, axis=0, keepdims=True)
    var = jnp.mean((h - mean) ** 2, axis=0, keepdims=True)
    t = jnp.maximum((h - mean) / jnp.sqrt(var + EPS), 0.0)
    o_ref[...] = jnp.dot(t, wl2_ref[...],
                         preferred_element_type=jnp.float32) + bl2_ref[...]


_tc_first = pl.pallas_call(
    _tc_first_body, out_shape=(jax.ShapeDtypeStruct((N, D), jnp.float32),
                               jax.ShapeDtypeStruct((N, 1), jnp.float32)))
_tc_mid = pl.pallas_call(
    _tc_mid_body, out_shape=jax.ShapeDtypeStruct((N, D), jnp.float32))
_tc_head = pl.pallas_call(
    _tc_head_body, out_shape=jax.ShapeDtypeStruct((N, 2), jnp.float32))


def kernel(x, edge_index, W0, b0, W1, b1, W2, b2, Wl1, bl1, Wl2, bl2):
    src = edge_index[0]
    dst = edge_index[1]
    pad = EPAD - E
    src_p = jnp.concatenate([src, jnp.zeros((pad,), jnp.int32)])
    dst_p = jnp.concatenate([dst, jnp.full((pad,), N, jnp.int32)])
    src_r = src_p.reshape(TOTSTEPS, CHUNK)
    dst_r = dst_p.reshape(TOTSTEPS, CHUNK)

    ones_nd = jnp.ones((N, D), jnp.float32)
    zeros = jnp.zeros((NPAD, D), jnp.float32)
    b0r, b1r, b2r = b0[None, :], b1[None, :], b2[None, :]
    bl1r, bl2r = bl1[None, :], bl2[None, :]

    sc_aggregate = _sc_aggregate_kernel()

    degS = sc_aggregate(ones_nd, src_r, dst_r, zeros)
    y, dinv = _tc_first(x, degS, W0)
    S = sc_aggregate(y, src_r, dst_r, zeros)
    y = _tc_mid(S, y, dinv, b0r, W1)
    S = sc_aggregate(y, src_r, dst_r, zeros)
    y = _tc_mid(S, y, dinv, b1r, W2)
    S = sc_aggregate(y, src_r, dst_r, zeros)
    out = _tc_head(S, y, dinv, b2r, Wl1, bl1r, Wl2, bl2r)
    return out


# speedup vs baseline: 1.2138x; 1.0420x over previous
import functools

import jax
import jax.numpy as jnp
from jax import lax
from jax.experimental import pallas as pl
from jax.experimental.pallas import tpu as pltpu, tpu_sc as plsc

N = 10000
D = 128
E = 320000
EPS = 1e-5

NC = 2
NS = 16
NW = NC * NS
CHUNK = 96
ST0 = 109
ST1 = 100
TOTSTEPS = NS * (ST0 + ST1)
EPAD = TOTSTEPS * CHUNK
RPT = -(-(N + 1) // (NS * 8)) * 8
NPAD = RPT * NS


@functools.cache
def _sc_mesh():
    return plsc.VectorSubcoreMesh(
        core_axis_name="c", subcore_axis_name="s",
        num_cores=NC, num_subcores=NS)


@functools.cache
def _sc_aggregate_kernel():
    return pl.kernel(
        _sc_aggregate_body,
        out_type=jax.ShapeDtypeStruct((NC, NPAD, D), jnp.float32),
        mesh=_sc_mesh(),
        scratch_types=[
            pltpu.VMEM((8, CHUNK), jnp.int32),
            pltpu.VMEM((8, CHUNK), jnp.int32),
            pltpu.VMEM((3, CHUNK, D), jnp.float32),
            pltpu.VMEM_SHARED((NPAD, D), jnp.float32),
            pltpu.SemaphoreType.DMA,
            pltpu.SemaphoreType.DMA,
            pltpu.SemaphoreType.DMA((3,)),
        ],
    )


def _sc_aggregate_body(y_hbm, src_hbm, dst_hbm, zeros_hbm, out_hbm,
                       src_v, dst_v, rows_v, acc_sh, isem, gsem, ssem):
    c = lax.axis_index("c")
    s = lax.axis_index("s")
    base = jnp.where(c == 0, s * ST0, NS * ST0 + s * ST1)
    nst = jnp.where(c == 0, ST0, ST1)

    pltpu.sync_copy(zeros_hbm.at[pl.ds(s * RPT, RPT)],
                    acc_sh.at[pl.ds(s * RPT, RPT)])
    pltpu.sync_copy(src_hbm.at[base], src_v.at[0])
    pltpu.sync_copy(dst_hbm.at[base], dst_v.at[0])
    pltpu.sync_copy(src_hbm.at[base + 1], src_v.at[1])
    pltpu.sync_copy(dst_hbm.at[base + 1], dst_v.at[1])
    plsc.subcore_barrier()
    pltpu.async_copy(y_hbm.at[src_v.at[0]], rows_v.at[0], gsem)

    def body(j, carry):
        buf = lax.rem(j, 3)

        @pl.when(j + 2 < nst)
        def _():
            slot = lax.rem(j + 2, 8)
            pltpu.async_copy(src_hbm.at[base + j + 2], src_v.at[slot], isem)
            pltpu.async_copy(dst_hbm.at[base + j + 2], dst_v.at[slot], isem)

        @pl.when(j >= 2)
        def _():
            jo = j - 2
            slot3 = lax.rem(jo, 3)
            pltpu.make_async_copy(
                rows_v.at[slot3],
                acc_sh.at[dst_v.at[lax.rem(jo, 8)]], ssem.at[slot3]).wait()

        pltpu.make_async_copy(y_hbm.at[src_v.at[lax.rem(j, 8)]],
                              rows_v.at[buf], gsem).wait()

        @pl.when(j + 1 < nst)
        def _():
            pltpu.async_copy(y_hbm.at[src_v.at[lax.rem(j + 1, 8)]],
                             rows_v.at[lax.rem(j + 1, 3)], gsem)

        pltpu.async_copy(rows_v.at[buf], acc_sh.at[dst_v.at[lax.rem(j, 8)]],
                         ssem.at[buf], add=True)

        @pl.when(j + 2 < nst)
        def _():
            slot = lax.rem(j + 2, 8)
            pltpu.make_async_copy(src_hbm.at[base + j + 2], src_v.at[slot],
                                  isem).wait()
            pltpu.make_async_copy(dst_hbm.at[base + j + 2], dst_v.at[slot],
                                  isem).wait()

        return carry

    lax.fori_loop(0, nst, body, 0)
    for dd in (2, 1):
        jo = nst - dd
        pltpu.make_async_copy(
            rows_v.at[lax.rem(jo, 3)],
            acc_sh.at[dst_v.at[lax.rem(jo, 8)]], ssem.at[lax.rem(jo, 3)]).wait()
    plsc.subcore_barrier()
    pltpu.sync_copy(acc_sh.at[pl.ds(s * RPT, RPT)],
                    out_hbm.at[c, pl.ds(s * RPT, RPT)])


def _tc_first_body(x_ref, degs_ref, w_ref, y_ref, dinv_ref):
    deg = degs_ref[0, :N, 0] + degs_ref[1, :N, 0] + 1.0
    dinv = lax.rsqrt(deg)
    dinv_ref[...] = dinv[:, None]
    y_ref[...] = dinv[:, None] * jnp.dot(x_ref[...], w_ref[...],
                                         preferred_element_type=jnp.float32)


def _tc_mid_body(s_ref, y_ref, dinv_ref, b_ref, w_ref, o_ref):
    dinv = dinv_ref[...]
    agg = s_ref[0, :N, :] + s_ref[1, :N, :] + y_ref[...]
    z = agg * dinv + b_ref[...]
    mean = jnp.mean(z, axis=0, keepdims=True)
    var = jnp.mean((z - mean) ** 2, axis=0, keepdims=True)
    t = jnp.maximum((z - mean) / jnp.sqrt(var + EPS), 0.0)
    o_ref[...] = dinv * jnp.dot(t, w_ref[...],
                                preferred_element_type=jnp.float32)


def _tc_head_body(s_ref, y_ref, dinv_ref, b_ref, wl1_ref, bl1_ref, wl2_ref,
                  bl2_ref, o_ref):
    dinv = dinv_ref[...]
    agg = s_ref[0, :N, :] + s_ref[1, :N, :] + y_ref[...]
    z = agg * dinv + b_ref[...]
    mean = jnp.mean(z, axis=0, keepdims=True)
    var = jnp.mean((z - mean) ** 2, axis=0, keepdims=True)
    t = jnp.maximum((z - mean) / jnp.sqrt(var + EPS), 0.0)
    h = jnp.dot(t, wl1_ref[...],
                preferred_element_type=jnp.float32) + bl1_ref[...]
    mean = jnp.mean(h, axis=0, keepdims=True)
    var = jnp.mean((h - mean) ** 2, axis=0, keepdims=True)
    t = jnp.maximum((h - mean) / jnp.sqrt(var + EPS), 0.0)
    o_ref[...] = jnp.dot(t, wl2_ref[...],
                         preferred_element_type=jnp.float32) + bl2_ref[...]


_tc_first = pl.pallas_call(
    _tc_first_body, out_shape=(jax.ShapeDtypeStruct((N, D), jnp.float32),
                               jax.ShapeDtypeStruct((N, 1), jnp.float32)))
_tc_mid = pl.pallas_call(
    _tc_mid_body, out_shape=jax.ShapeDtypeStruct((N, D), jnp.float32))
_tc_head = pl.pallas_call(
    _tc_head_body, out_shape=jax.ShapeDtypeStruct((N, 2), jnp.float32))


def kernel(x, edge_index, W0, b0, W1, b1, W2, b2, Wl1, bl1, Wl2, bl2):
    src = edge_index[0]
    dst = edge_index[1]
    pad = EPAD - E
    src_p = jnp.concatenate([src, jnp.zeros((pad,), jnp.int32)])
    dst_p = jnp.concatenate([dst, jnp.full((pad,), N, jnp.int32)])
    src_r = src_p.reshape(TOTSTEPS, CHUNK)
    dst_r = dst_p.reshape(TOTSTEPS, CHUNK)

    ones_nd = jnp.ones((N, D), jnp.float32)
    zeros = jnp.zeros((NPAD, D), jnp.float32)
    b0r, b1r, b2r = b0[None, :], b1[None, :], b2[None, :]
    bl1r, bl2r = bl1[None, :], bl2[None, :]

    sc_aggregate = _sc_aggregate_kernel()

    degS = sc_aggregate(ones_nd, src_r, dst_r, zeros)
    y, dinv = _tc_first(x, degS, W0)
    S = sc_aggregate(y, src_r, dst_r, zeros)
    y = _tc_mid(S, y, dinv, b0r, W1)
    S = sc_aggregate(y, src_r, dst_r, zeros)
    y = _tc_mid(S, y, dinv, b1r, W2)
    S = sc_aggregate(y, src_r, dst_r, zeros)
    out = _tc_head(S, y, dinv, b2r, Wl1, bl1r, Wl2, bl2r)
    return out
